# baseline (speedup 1.0000x reference)
import functools

import jax
import jax.numpy as jnp
from jax import lax
from jax.experimental import pallas as pl
from jax.experimental.pallas import tpu as pltpu
from jax.experimental.pallas import tpu_sc as plsc

B_ROWS = 4
B_COLS = 8192
D_MODEL = 512
NUM_WORKERS = 32
BPW = B_ROWS * B_COLS // NUM_WORKERS
TPR = B_COLS // BPW
CHUNK = 64
NCHUNKS = BPW // CHUNK
NBUF = 3

_mesh = plsc.VectorSubcoreMesh(core_axis_name="c", subcore_axis_name="s")


@functools.partial(
    pl.kernel,
    mesh=_mesh,
    out_type=jax.ShapeDtypeStruct((B_ROWS, B_COLS, D_MODEL), jnp.float32),
    scratch_types=[
        pltpu.VMEM((BPW,), jnp.int32),
        *[pltpu.VMEM((CHUNK, D_MODEL), jnp.float32) for _ in range(NBUF)],
        *[pltpu.SemaphoreType.DMA for _ in range(2 * NBUF)],
    ],
)
def _embed_gather(idx_hbm, table_hbm, out_hbm, idx_v, *bufs_and_sems):
    bufs = bufs_and_sems[:NBUF]
    gsems = bufs_and_sems[NBUF:2 * NBUF]
    wsems = bufs_and_sems[2 * NBUF:]
    wid = lax.axis_index("s") * 2 + lax.axis_index("c")
    row = wid // TPR
    col0 = (wid % TPR) * BPW
    pltpu.sync_copy(idx_hbm.at[row, pl.ds(col0, BPW)], idx_v)

    def start_gather(c):
        b = c % NBUF
        idx_slice = idx_v.at[pl.ds(c * CHUNK, CHUNK)]
        return pltpu.async_copy(table_hbm.at[idx_slice], bufs[b], gsems[b])

    def start_write(c):
        b = c % NBUF
        dst = out_hbm.at[row, pl.ds(col0 + c * CHUNK, CHUNK)]
        return pltpu.async_copy(bufs[b], dst, wsems[b])

    gathers = [None] * NCHUNKS
    writes = [None] * NCHUNKS
    for c in range(NBUF - 1):
        gathers[c] = start_gather(c)
    for c in range(NCHUNKS):
        if c > 0:
            writes[c - 1].wait()
        g = c + NBUF - 1
        if g < NCHUNKS:
            gathers[g] = start_gather(g)
        gathers[c].wait()
        writes[c] = start_write(c)
    writes[NCHUNKS - 1].wait()


def kernel(x, table):
    return _embed_gather(x.astype(jnp.int32), table)


# speedup vs baseline: 1.0084x; 1.0084x over previous
import functools

import jax
import jax.numpy as jnp
from jax import lax
from jax.experimental import pallas as pl
from jax.experimental.pallas import tpu as pltpu
from jax.experimental.pallas import tpu_sc as plsc

B_ROWS = 4
B_COLS = 8192
D_MODEL = 512
NUM_WORKERS = 32
BPW = B_ROWS * B_COLS // NUM_WORKERS
TPR = B_COLS // BPW
CHUNKS = [80] * 12 + [64]
OFFS = [sum(CHUNKS[:i]) for i in range(len(CHUNKS))]
NCHUNKS = len(CHUNKS)
CHUNK_MAX = 80
NBUF = 3

_mesh = plsc.VectorSubcoreMesh(core_axis_name="c", subcore_axis_name="s")


@functools.partial(
    pl.kernel,
    mesh=_mesh,
    out_type=jax.ShapeDtypeStruct((B_ROWS, B_COLS, D_MODEL), jnp.float32),
    scratch_types=[
        pltpu.VMEM((BPW,), jnp.int32),
        *[pltpu.VMEM((CHUNK_MAX, D_MODEL), jnp.float32) for _ in range(NBUF)],
        *[pltpu.SemaphoreType.DMA for _ in range(2 * NBUF)],
    ],
)
def _embed_gather(idx_hbm, table_hbm, out_hbm, idx_v, *bufs_and_sems):
    bufs = bufs_and_sems[:NBUF]
    gsems = bufs_and_sems[NBUF:2 * NBUF]
    wsems = bufs_and_sems[2 * NBUF:]
    wid = lax.axis_index("s") * 2 + lax.axis_index("c")
    row = wid // TPR
    col0 = (wid % TPR) * BPW
    pltpu.sync_copy(idx_hbm.at[row, pl.ds(col0, BPW)], idx_v)

    def start_gather(c):
        b = c % NBUF
        n = CHUNKS[c]
        idx_slice = idx_v.at[pl.ds(OFFS[c], n)]
        return pltpu.async_copy(table_hbm.at[idx_slice],
                                bufs[b].at[pl.ds(0, n)], gsems[b])

    def start_write(c):
        b = c % NBUF
        n = CHUNKS[c]
        dst = out_hbm.at[row, pl.ds(col0 + OFFS[c], n)]
        return pltpu.async_copy(bufs[b].at[pl.ds(0, n)], dst, wsems[b])

    gathers = [None] * NCHUNKS
    writes = [None] * NCHUNKS
    for c in range(NBUF - 1):
        gathers[c] = start_gather(c)
    for c in range(NCHUNKS):
        if c > 0:
            writes[c - 1].wait()
        g = c + NBUF - 1
        if g < NCHUNKS:
            gathers[g] = start_gather(g)
        gathers[c].wait()
        writes[c] = start_write(c)
    writes[NCHUNKS - 1].wait()


def kernel(x, table):
    return _embed_gather(x.astype(jnp.int32), table)
